# baseline (speedup 1.0000x reference)
import functools

import jax
import jax.numpy as jnp
from jax import lax
from jax.experimental import pallas as pl
from jax.experimental.pallas import tpu as pltpu
from jax.experimental.pallas import tpu_sc as plsc

_N = 10000
_E = 320000
_H = 128
_G = 64
_C = 10
_L = 4

_NC = 2
_NS = 16
_NW = _NC * _NS
_EPW = _E // _NW
_EC = 64
_NCH = 157
_PEPW = _NCH * _EC
_NP = 10240
_RPT = _NP // _NS


def _agg_body(h_hbm, idx_hbm, zeros_hbm, out_hbm,
              idx_v, rows0_v, rows1_v, acc_sh, sem0, sem1):
    cid = lax.axis_index("c")
    sid = lax.axis_index("s")
    wid = sid * _NC + cid
    pltpu.sync_copy(idx_hbm.at[wid], idx_v)
    gather0 = pltpu.async_copy(h_hbm.at[idx_v.at[0, pl.ds(0, _EC)]],
                               rows0_v, sem0)
    gather1 = pltpu.async_copy(h_hbm.at[idx_v.at[1, pl.ds(0, _EC)]],
                               rows1_v, sem1)
    pltpu.sync_copy(zeros_hbm.at[pl.ds(sid * _RPT, _RPT)],
                    acc_sh.at[pl.ds(sid * _RPT, _RPT)])
    plsc.subcore_barrier()

    def step(i, carry):
        c = 2 * i
        gather0.wait()
        pltpu.sync_copy(rows0_v, acc_sh.at[idx_v.at[c, pl.ds(_EC, _EC)]],
                        add=True)
        pltpu.async_copy(h_hbm.at[idx_v.at[c + 2, pl.ds(0, _EC)]],
                         rows0_v, sem0)
        gather1.wait()
        pltpu.sync_copy(rows1_v, acc_sh.at[idx_v.at[c + 1, pl.ds(_EC, _EC)]],
                        add=True)
        pltpu.async_copy(
            h_hbm.at[idx_v.at[jnp.minimum(c + 3, _NCH - 1), pl.ds(0, _EC)]],
            rows1_v, sem1)
        return carry

    lax.fori_loop(0, (_NCH - 1) // 2, step, 0)
    gather0.wait()
    pltpu.sync_copy(rows0_v, acc_sh.at[idx_v.at[_NCH - 1, pl.ds(_EC, _EC)]],
                    add=True)
    gather1.wait()
    plsc.subcore_barrier()
    pltpu.sync_copy(acc_sh.at[pl.ds(sid * _RPT, _RPT)],
                    out_hbm.at[cid, pl.ds(sid * _RPT, _RPT)])


_agg_call = pl.kernel(
    _agg_body,
    out_type=jax.ShapeDtypeStruct((_NC, _NP, _H), jnp.float32),
    mesh=plsc.VectorSubcoreMesh(core_axis_name="c", subcore_axis_name="s"),
    scratch_types=[
        pltpu.VMEM((_NCH, 2 * _EC), jnp.int32),
        pltpu.VMEM((_EC, _H), jnp.float32),
        pltpu.VMEM((_EC, _H), jnp.float32),
        pltpu.VMEM_SHARED((_NP, _H), jnp.float32),
        pltpu.SemaphoreType.DMA,
        pltpu.SemaphoreType.DMA,
    ],
)


def _bn_relu(y, g, t):
    m = jnp.mean(y, axis=0, keepdims=True)
    d = y - m
    v = jnp.mean(d * d, axis=0, keepdims=True)
    return jnp.maximum(d * lax.rsqrt(v + 1e-5) * g + t, 0.0)


def _mlp_core(h_ref, p_ref, w1_ref, b1_ref, g1_ref, t1_ref,
              w2_ref, b2_ref, g2_ref, t2_ref):
    z = h_ref[...] + p_ref[0] + p_ref[1]
    y = jnp.dot(z, w1_ref[...], preferred_element_type=jnp.float32,
                precision=lax.Precision.HIGHEST) + b1_ref[...]
    y = _bn_relu(y, g1_ref[...], t1_ref[...])
    y = jnp.dot(y, w2_ref[...], preferred_element_type=jnp.float32,
                precision=lax.Precision.HIGHEST) + b2_ref[...]
    return _bn_relu(y, g2_ref[...], t2_ref[...])


def _mlp_body(h_ref, p_ref, w1_ref, b1_ref, g1_ref, t1_ref,
              w2_ref, b2_ref, g2_ref, t2_ref, o_ref):
    o_ref[...] = _mlp_core(h_ref, p_ref, w1_ref, b1_ref, g1_ref, t1_ref,
                           w2_ref, b2_ref, g2_ref, t2_ref)


def _mlp_tail_body(h_ref, p_ref, w1_ref, b1_ref, g1_ref, t1_ref,
                   w2_ref, b2_ref, g2_ref, t2_ref,
                   batch_ref, l1w_ref, l1b_ref, l2w_ref, l2b_ref, o_ref):
    h4 = _mlp_core(h_ref, p_ref, w1_ref, b1_ref, g1_ref, t1_ref,
                   w2_ref, b2_ref, g2_ref, t2_ref)
    gids = lax.broadcasted_iota(jnp.int32, (_G, _N), 0)
    mask = (batch_ref[...] == gids).astype(jnp.float32)
    pooled = jnp.dot(mask, h4, preferred_element_type=jnp.float32,
                     precision=lax.Precision.HIGHEST)
    t = jnp.maximum(jnp.dot(pooled, l1w_ref[...],
                            preferred_element_type=jnp.float32,
                            precision=lax.Precision.HIGHEST)
                    + l1b_ref[...], 0.0)
    o_ref[...] = jnp.dot(t, l2w_ref[...], preferred_element_type=jnp.float32,
                         precision=lax.Precision.HIGHEST) + l2b_ref[...]


_mlp_call = pl.pallas_call(
    _mlp_body, out_shape=jax.ShapeDtypeStruct((_N, _H), jnp.float32))

_mlp_tail_call = pl.pallas_call(
    _mlp_tail_body, out_shape=jax.ShapeDtypeStruct((_G, _C), jnp.float32))


def kernel(x, W1s, b1s, g1s, bt1s, W2s, b2s, g2s, bt2s,
           lin1_W, lin1_b, lin2_W, lin2_b, edge_index, batch):
    pad = _PEPW - _EPW
    srcs = jnp.pad(edge_index[0].reshape(_NW, _EPW), ((0, 0), (0, pad)))
    dsts = jnp.pad(edge_index[1].reshape(_NW, _EPW), ((0, 0), (0, pad)),
                   constant_values=_N)
    idx = jnp.concatenate([srcs.reshape(_NW, _NCH, _EC),
                           dsts.reshape(_NW, _NCH, _EC)], axis=-1)
    zeros = jnp.zeros((_NP, _H), jnp.float32)
    row = lambda a: a.reshape(1, -1)

    h = x
    for i in range(_L):
        parts = _agg_call(h, idx, zeros)[:, :_N, :]
        args = (h, parts, W1s[i], row(b1s[i]), row(g1s[i]), row(bt1s[i]),
                W2s[i], row(b2s[i]), row(g2s[i]), row(bt2s[i]))
        if i < _L - 1:
            h = _mlp_call(*args)
        else:
            out = _mlp_tail_call(*args, row(batch), lin1_W, row(lin1_b),
                                 lin2_W, row(lin2_b))
    return out


# speedup vs baseline: 1.2536x; 1.2536x over previous
import functools

import jax
import jax.numpy as jnp
from jax import lax
from jax.experimental import pallas as pl
from jax.experimental.pallas import tpu as pltpu
from jax.experimental.pallas import tpu_sc as plsc

_N = 10000
_E = 320000
_H = 128
_G = 64
_C = 10
_L = 4

_NC = 2
_NS = 16
_NW = _NC * _NS
_EPW = _E // _NW
_EC = 64
_NCH = 157
_PEPW = _NCH * _EC
_NP = 10240
_RPT = _NP // _NS


def _agg_body(h_hbm, idx_hbm, zeros_hbm, out_hbm,
              idx_v, rows0_v, rows1_v, acc_sh, sem0, sem1):
    cid = lax.axis_index("c")
    sid = lax.axis_index("s")
    wid = sid * _NC + cid
    pltpu.sync_copy(idx_hbm.at[wid], idx_v)
    gather0 = pltpu.async_copy(h_hbm.at[idx_v.at[0, pl.ds(0, _EC)]],
                               rows0_v, sem0)
    pltpu.sync_copy(zeros_hbm.at[pl.ds(sid * _RPT, _RPT)],
                    acc_sh.at[pl.ds(sid * _RPT, _RPT)])
    plsc.subcore_barrier()

    def step(i, carry):
        c = 2 * i
        gather1 = pltpu.async_copy(h_hbm.at[idx_v.at[c + 1, pl.ds(0, _EC)]],
                                   rows1_v, sem1)
        gather0.wait()
        pltpu.sync_copy(rows0_v, acc_sh.at[idx_v.at[c, pl.ds(_EC, _EC)]],
                        add=True)
        pltpu.async_copy(h_hbm.at[idx_v.at[c + 2, pl.ds(0, _EC)]],
                         rows0_v, sem0)
        gather1.wait()
        pltpu.sync_copy(rows1_v, acc_sh.at[idx_v.at[c + 1, pl.ds(_EC, _EC)]],
                        add=True)
        return carry

    lax.fori_loop(0, (_NCH - 1) // 2, step, 0)
    gather0.wait()
    pltpu.sync_copy(rows0_v, acc_sh.at[idx_v.at[_NCH - 1, pl.ds(_EC, _EC)]],
                    add=True)
    plsc.subcore_barrier()
    pltpu.sync_copy(acc_sh.at[pl.ds(sid * _RPT, _RPT)],
                    out_hbm.at[cid, pl.ds(sid * _RPT, _RPT)])


_agg_call = pl.kernel(
    _agg_body,
    out_type=jax.ShapeDtypeStruct((_NC, _NP, _H), jnp.float32),
    mesh=plsc.VectorSubcoreMesh(core_axis_name="c", subcore_axis_name="s"),
    scratch_types=[
        pltpu.VMEM((_NCH, 2 * _EC), jnp.int32),
        pltpu.VMEM((_EC, _H), jnp.float32),
        pltpu.VMEM((_EC, _H), jnp.float32),
        pltpu.VMEM_SHARED((_NP, _H), jnp.float32),
        pltpu.SemaphoreType.DMA,
        pltpu.SemaphoreType.DMA,
    ],
)


def _bn_relu(y, g, t):
    m = jnp.mean(y, axis=0, keepdims=True)
    d = y - m
    v = jnp.mean(d * d, axis=0, keepdims=True)
    return jnp.maximum(d * lax.rsqrt(v + 1e-5) * g + t, 0.0)


def _mlp_core(h_ref, p_ref, w1_ref, b1_ref, g1_ref, t1_ref,
              w2_ref, b2_ref, g2_ref, t2_ref):
    z = h_ref[...] + p_ref[0] + p_ref[1]
    y = jnp.dot(z, w1_ref[...], preferred_element_type=jnp.float32,
                precision=lax.Precision.HIGHEST) + b1_ref[...]
    y = _bn_relu(y, g1_ref[...], t1_ref[...])
    y = jnp.dot(y, w2_ref[...], preferred_element_type=jnp.float32,
                precision=lax.Precision.HIGHEST) + b2_ref[...]
    return _bn_relu(y, g2_ref[...], t2_ref[...])


def _mlp_body(h_ref, p_ref, w1_ref, b1_ref, g1_ref, t1_ref,
              w2_ref, b2_ref, g2_ref, t2_ref, o_ref):
    o_ref[...] = _mlp_core(h_ref, p_ref, w1_ref, b1_ref, g1_ref, t1_ref,
                           w2_ref, b2_ref, g2_ref, t2_ref)


def _mlp_tail_body(h_ref, p_ref, w1_ref, b1_ref, g1_ref, t1_ref,
                   w2_ref, b2_ref, g2_ref, t2_ref,
                   batch_ref, l1w_ref, l1b_ref, l2w_ref, l2b_ref, o_ref):
    h4 = _mlp_core(h_ref, p_ref, w1_ref, b1_ref, g1_ref, t1_ref,
                   w2_ref, b2_ref, g2_ref, t2_ref)
    gids = lax.broadcasted_iota(jnp.int32, (_G, _N), 0)
    mask = (batch_ref[...] == gids).astype(jnp.float32)
    pooled = jnp.dot(mask, h4, preferred_element_type=jnp.float32,
                     precision=lax.Precision.HIGHEST)
    t = jnp.maximum(jnp.dot(pooled, l1w_ref[...],
                            preferred_element_type=jnp.float32,
                            precision=lax.Precision.HIGHEST)
                    + l1b_ref[...], 0.0)
    o_ref[...] = jnp.dot(t, l2w_ref[...], preferred_element_type=jnp.float32,
                         precision=lax.Precision.HIGHEST) + l2b_ref[...]


_mlp_call = pl.pallas_call(
    _mlp_body, out_shape=jax.ShapeDtypeStruct((_N, _H), jnp.float32))

_mlp_tail_call = pl.pallas_call(
    _mlp_tail_body, out_shape=jax.ShapeDtypeStruct((_G, _C), jnp.float32))


def kernel(x, W1s, b1s, g1s, bt1s, W2s, b2s, g2s, bt2s,
           lin1_W, lin1_b, lin2_W, lin2_b, edge_index, batch):
    pad = _PEPW - _EPW
    srcs = jnp.pad(edge_index[0].reshape(_NW, _EPW), ((0, 0), (0, pad)))
    dsts = jnp.pad(edge_index[1].reshape(_NW, _EPW), ((0, 0), (0, pad)),
                   constant_values=_N)
    idx = jnp.concatenate([srcs.reshape(_NW, _NCH, _EC),
                           dsts.reshape(_NW, _NCH, _EC)], axis=-1)
    zeros = jnp.zeros((_NP, _H), jnp.float32)
    row = lambda a: a.reshape(1, -1)

    h = x
    for i in range(_L):
        parts = _agg_call(h, idx, zeros)[:, :_N, :]
        args = (h, parts, W1s[i], row(b1s[i]), row(g1s[i]), row(bt1s[i]),
                W2s[i], row(b2s[i]), row(g2s[i]), row(bt2s[i]))
        if i < _L - 1:
            h = _mlp_call(*args)
        else:
            out = _mlp_tail_call(*args, row(batch), lin1_W, row(lin1_b),
                                 lin2_W, row(lin2_b))
    return out


# speedup vs baseline: 1.2949x; 1.0329x over previous
import functools

import jax
import jax.numpy as jnp
from jax import lax
from jax.experimental import pallas as pl
from jax.experimental.pallas import tpu as pltpu
from jax.experimental.pallas import tpu_sc as plsc

_N = 10000
_E = 320000
_H = 128
_G = 64
_C = 10
_L = 4

_NC = 2
_NS = 16
_NW = _NC * _NS
_EPW = _E // _NW
_EC = 64
_NCH = 157
_PEPW = _NCH * _EC
_NP = 10240
_RPT = _NP // _NS


def _agg_body(h_hbm, idx_hbm, zeros_hbm, out_hbm,
              idx_v, rows0_v, rows1_v, acc_sh, sem0, sem1):
    cid = lax.axis_index("c")
    sid = lax.axis_index("s")
    wid = sid * _NC + cid
    pltpu.sync_copy(idx_hbm.at[wid], idx_v)
    gather0 = pltpu.async_copy(h_hbm.at[idx_v.at[0, pl.ds(0, _EC)]],
                               rows0_v, sem0)
    pltpu.sync_copy(zeros_hbm.at[pl.ds(sid * _RPT, _RPT)],
                    acc_sh.at[pl.ds(sid * _RPT, _RPT)])
    plsc.subcore_barrier()

    def step(i, carry):
        c = 2 * i
        gather1 = pltpu.async_copy(h_hbm.at[idx_v.at[c + 1, pl.ds(0, _EC)]],
                                   rows1_v, sem1)
        gather0.wait()
        pltpu.sync_copy(rows0_v, acc_sh.at[idx_v.at[c, pl.ds(_EC, _EC)]],
                        add=True)
        pltpu.async_copy(h_hbm.at[idx_v.at[c + 2, pl.ds(0, _EC)]],
                         rows0_v, sem0)
        gather1.wait()
        pltpu.sync_copy(rows1_v, acc_sh.at[idx_v.at[c + 1, pl.ds(_EC, _EC)]],
                        add=True)
        return carry

    lax.fori_loop(0, (_NCH - 1) // 2, step, 0)
    gather0.wait()
    pltpu.sync_copy(rows0_v, acc_sh.at[idx_v.at[_NCH - 1, pl.ds(_EC, _EC)]],
                    add=True)
    plsc.subcore_barrier()
    pltpu.sync_copy(acc_sh.at[pl.ds(sid * _RPT, _RPT)],
                    out_hbm.at[cid, pl.ds(sid * _RPT, _RPT)])


_agg_call = pl.kernel(
    _agg_body,
    out_type=jax.ShapeDtypeStruct((_NC, _NP, _H), jnp.float32),
    mesh=plsc.VectorSubcoreMesh(core_axis_name="c", subcore_axis_name="s"),
    scratch_types=[
        pltpu.VMEM((_NCH, 2 * _EC), jnp.int32),
        pltpu.VMEM((_EC, _H), jnp.float32),
        pltpu.VMEM((_EC, _H), jnp.float32),
        pltpu.VMEM_SHARED((_NP, _H), jnp.float32),
        pltpu.SemaphoreType.DMA,
        pltpu.SemaphoreType.DMA,
    ],
)


def _bn_relu(y, g, t):
    m = jnp.mean(y, axis=0, keepdims=True)
    d = y - m
    v = jnp.mean(d * d, axis=0, keepdims=True)
    return jnp.maximum(d * lax.rsqrt(v + 1e-5) * g + t, 0.0)


def _mlp_core(h_ref, p_ref, w1_ref, b1_ref, g1_ref, t1_ref,
              w2_ref, b2_ref, g2_ref, t2_ref):
    z = h_ref[...] + p_ref[0, :_N] + p_ref[1, :_N]
    y = jnp.dot(z, w1_ref[...], preferred_element_type=jnp.float32,
                precision=lax.Precision.HIGHEST) + b1_ref[...]
    y = _bn_relu(y, g1_ref[...], t1_ref[...])
    y = jnp.dot(y, w2_ref[...], preferred_element_type=jnp.float32,
                precision=lax.Precision.HIGHEST) + b2_ref[...]
    return _bn_relu(y, g2_ref[...], t2_ref[...])


def _mlp_body(h_ref, p_ref, w1_ref, b1_ref, g1_ref, t1_ref,
              w2_ref, b2_ref, g2_ref, t2_ref, o_ref):
    o_ref[...] = _mlp_core(h_ref, p_ref, w1_ref, b1_ref, g1_ref, t1_ref,
                           w2_ref, b2_ref, g2_ref, t2_ref)


def _mlp_tail_body(h_ref, p_ref, w1_ref, b1_ref, g1_ref, t1_ref,
                   w2_ref, b2_ref, g2_ref, t2_ref,
                   batch_ref, l1w_ref, l1b_ref, l2w_ref, l2b_ref, o_ref):
    h4 = _mlp_core(h_ref, p_ref, w1_ref, b1_ref, g1_ref, t1_ref,
                   w2_ref, b2_ref, g2_ref, t2_ref)
    gids = lax.broadcasted_iota(jnp.int32, (_G, _N), 0)
    mask = (batch_ref[...] == gids).astype(jnp.float32)
    pooled = jnp.dot(mask, h4, preferred_element_type=jnp.float32,
                     precision=lax.Precision.HIGHEST)
    t = jnp.maximum(jnp.dot(pooled, l1w_ref[...],
                            preferred_element_type=jnp.float32,
                            precision=lax.Precision.HIGHEST)
                    + l1b_ref[...], 0.0)
    o_ref[...] = jnp.dot(t, l2w_ref[...], preferred_element_type=jnp.float32,
                         precision=lax.Precision.HIGHEST) + l2b_ref[...]


_mlp_call = pl.pallas_call(
    _mlp_body, out_shape=jax.ShapeDtypeStruct((_N, _H), jnp.float32))

_mlp_tail_call = pl.pallas_call(
    _mlp_tail_body, out_shape=jax.ShapeDtypeStruct((_G, _C), jnp.float32))


def kernel(x, W1s, b1s, g1s, bt1s, W2s, b2s, g2s, bt2s,
           lin1_W, lin1_b, lin2_W, lin2_b, edge_index, batch):
    pad = _PEPW - _EPW
    srcs = jnp.pad(edge_index[0].reshape(_NW, _EPW), ((0, 0), (0, pad)))
    dsts = jnp.pad(edge_index[1].reshape(_NW, _EPW), ((0, 0), (0, pad)),
                   constant_values=_N)
    idx = jnp.concatenate([srcs.reshape(_NW, _NCH, _EC),
                           dsts.reshape(_NW, _NCH, _EC)], axis=-1)
    zeros = jnp.zeros((_NP, _H), jnp.float32)
    row = lambda a: a.reshape(1, -1)

    h = x
    for i in range(_L):
        parts = _agg_call(h, idx, zeros)
        args = (h, parts, W1s[i], row(b1s[i]), row(g1s[i]), row(bt1s[i]),
                W2s[i], row(b2s[i]), row(g2s[i]), row(bt2s[i]))
        if i < _L - 1:
            h = _mlp_call(*args)
        else:
            out = _mlp_tail_call(*args, row(batch), lin1_W, row(lin1_b),
                                 lin2_W, row(lin2_b))
    return out


# speedup vs baseline: 1.3178x; 1.0177x over previous
import functools

import jax
import jax.numpy as jnp
from jax import lax
from jax.experimental import pallas as pl
from jax.experimental.pallas import tpu as pltpu
from jax.experimental.pallas import tpu_sc as plsc

_N = 10000
_E = 320000
_H = 128
_G = 64
_C = 10
_L = 4

_NC = 2
_NS = 16
_NW = _NC * _NS
_EPW = _E // _NW
_EC = 64
_NCH = 157
_PEPW = _NCH * _EC
_NP = 10240
_RPT = _NP // _NS


def _agg_body(h_hbm, idx_hbm, zeros_hbm, out_hbm,
              idx_v, rows0_v, rows1_v, acc_sh, sem0, sem1):
    cid = lax.axis_index("c")
    sid = lax.axis_index("s")
    wid = sid * _NC + cid
    pltpu.sync_copy(idx_hbm.at[wid], idx_v)
    gather0 = pltpu.async_copy(h_hbm.at[idx_v.at[0, pl.ds(0, _EC)]],
                               rows0_v, sem0)
    pltpu.sync_copy(zeros_hbm.at[pl.ds(sid * _RPT, _RPT)],
                    acc_sh.at[pl.ds(sid * _RPT, _RPT)])
    plsc.subcore_barrier()

    def step(i, carry):
        c = 2 * i
        gather1 = pltpu.async_copy(h_hbm.at[idx_v.at[c + 1, pl.ds(0, _EC)]],
                                   rows1_v, sem1)
        gather0.wait()
        pltpu.sync_copy(rows0_v, acc_sh.at[idx_v.at[c, pl.ds(_EC, _EC)]],
                        add=True)
        pltpu.async_copy(h_hbm.at[idx_v.at[c + 2, pl.ds(0, _EC)]],
                         rows0_v, sem0)
        gather1.wait()
        pltpu.sync_copy(rows1_v, acc_sh.at[idx_v.at[c + 1, pl.ds(_EC, _EC)]],
                        add=True)
        return carry

    lax.fori_loop(0, (_NCH - 1) // 2, step, 0)
    gather0.wait()
    pltpu.sync_copy(rows0_v, acc_sh.at[idx_v.at[_NCH - 1, pl.ds(_EC, _EC)]],
                    add=True)
    plsc.subcore_barrier()
    pltpu.sync_copy(acc_sh.at[pl.ds(sid * _RPT, _RPT)],
                    out_hbm.at[cid, pl.ds(sid * _RPT, _RPT)])


_agg_call = pl.kernel(
    _agg_body,
    out_type=jax.ShapeDtypeStruct((_NC, _NP, _H), jnp.float32),
    mesh=plsc.VectorSubcoreMesh(core_axis_name="c", subcore_axis_name="s"),
    scratch_types=[
        pltpu.VMEM((_NCH, 2 * _EC), jnp.int32),
        pltpu.VMEM((_EC, _H), jnp.float32),
        pltpu.VMEM((_EC, _H), jnp.float32),
        pltpu.VMEM_SHARED((_NP, _H), jnp.float32),
        pltpu.SemaphoreType.DMA,
        pltpu.SemaphoreType.DMA,
    ],
)


def _bn_relu(y, g, t):
    m = jnp.mean(y, axis=0, keepdims=True)
    ms = jnp.mean(y * y, axis=0, keepdims=True)
    a = lax.rsqrt(ms - m * m + 1e-5) * g
    return jnp.maximum(y * a + (t - m * a), 0.0)


def _mlp_core(h_ref, p_ref, w1_ref, b1_ref, g1_ref, t1_ref,
              w2_ref, b2_ref, g2_ref, t2_ref):
    z = h_ref[...] + p_ref[0, :_N] + p_ref[1, :_N]
    y = jnp.dot(z, w1_ref[...], preferred_element_type=jnp.float32,
                precision=lax.Precision.HIGHEST) + b1_ref[...]
    y = _bn_relu(y, g1_ref[...], t1_ref[...])
    y = jnp.dot(y, w2_ref[...], preferred_element_type=jnp.float32,
                precision=lax.Precision.HIGHEST) + b2_ref[...]
    return _bn_relu(y, g2_ref[...], t2_ref[...])


def _mlp_body(h_ref, p_ref, w1_ref, b1_ref, g1_ref, t1_ref,
              w2_ref, b2_ref, g2_ref, t2_ref, o_ref):
    o_ref[...] = _mlp_core(h_ref, p_ref, w1_ref, b1_ref, g1_ref, t1_ref,
                           w2_ref, b2_ref, g2_ref, t2_ref)


def _mlp_tail_body(h_ref, p_ref, w1_ref, b1_ref, g1_ref, t1_ref,
                   w2_ref, b2_ref, g2_ref, t2_ref,
                   batch_ref, l1w_ref, l1b_ref, l2w_ref, l2b_ref, o_ref):
    h4 = _mlp_core(h_ref, p_ref, w1_ref, b1_ref, g1_ref, t1_ref,
                   w2_ref, b2_ref, g2_ref, t2_ref)
    gids = lax.broadcasted_iota(jnp.int32, (_G, _N), 0)
    mask = (batch_ref[...] == gids).astype(jnp.float32)
    pooled = jnp.dot(mask, h4, preferred_element_type=jnp.float32,
                     precision=lax.Precision.HIGHEST)
    t = jnp.maximum(jnp.dot(pooled, l1w_ref[...],
                            preferred_element_type=jnp.float32,
                            precision=lax.Precision.HIGHEST)
                    + l1b_ref[...], 0.0)
    o_ref[...] = jnp.dot(t, l2w_ref[...], preferred_element_type=jnp.float32,
                         precision=lax.Precision.HIGHEST) + l2b_ref[...]


_mlp_call = pl.pallas_call(
    _mlp_body, out_shape=jax.ShapeDtypeStruct((_N, _H), jnp.float32))

_mlp_tail_call = pl.pallas_call(
    _mlp_tail_body, out_shape=jax.ShapeDtypeStruct((_G, _C), jnp.float32))


def kernel(x, W1s, b1s, g1s, bt1s, W2s, b2s, g2s, bt2s,
           lin1_W, lin1_b, lin2_W, lin2_b, edge_index, batch):
    pad = _PEPW - _EPW
    srcs = jnp.pad(edge_index[0].reshape(_NW, _EPW), ((0, 0), (0, pad)))
    dsts = jnp.pad(edge_index[1].reshape(_NW, _EPW), ((0, 0), (0, pad)),
                   constant_values=_N)
    idx = jnp.concatenate([srcs.reshape(_NW, _NCH, _EC),
                           dsts.reshape(_NW, _NCH, _EC)], axis=-1)
    zeros = jnp.zeros((_NP, _H), jnp.float32)
    row = lambda a: a.reshape(1, -1)

    h = x
    for i in range(_L):
        parts = _agg_call(h, idx, zeros)
        args = (h, parts, W1s[i], row(b1s[i]), row(g1s[i]), row(bt1s[i]),
                W2s[i], row(b2s[i]), row(g2s[i]), row(bt2s[i]))
        if i < _L - 1:
            h = _mlp_call(*args)
        else:
            out = _mlp_tail_call(*args, row(batch), lin1_W, row(lin1_b),
                                 lin2_W, row(lin2_b))
    return out
